# baseline (speedup 1.0000x reference)
import functools

import jax
import jax.numpy as jnp
from jax import lax
from jax.experimental import pallas as pl
from jax.experimental.pallas import tpu as pltpu
from jax.experimental.pallas import tpu_sc as plsc

N, E, NDIM, HDL, R = 10000, 320000, 128, 256, 128
BN_EPS = 1e-5

NC, NS = 2, 16
NW = NC * NS
CHUNK = 40
EPW = E // NW
NCHUNK = EPW // CHUNK
NGRP = 10
GC = NCHUNK // NGRP
NPAD = 10240
RPS = NPAD // NS
ZROWS = 8

BLK = 1000


_SCATTER_ON = True
_GDIM = NDIM
_GDT = jnp.float32
NBUF = 7
DEPTH = NBUF - 2


def _sc_agg_body(x_hbm, sd_hbm, out_hbm, sd_v, rows_v, zero_v, acc_sh, sems,
                 ssems):
    c = lax.axis_index("c")
    s = lax.axis_index("s")
    w = c * NS + s

    zf = jnp.zeros((16,), jnp.float32)

    def _zfill(i, carry):
        r = i // (NDIM // 16)
        k = i % (NDIM // 16)
        zero_v[r, pl.ds(k * 16, 16)] = zf
        return carry

    lax.fori_loop(0, ZROWS * (NDIM // 16), _zfill, 0)

    base = s * RPS

    def _zcopy(t, carry):
        pltpu.sync_copy(zero_v, acc_sh.at[pl.ds(base + t * ZROWS, ZROWS)])
        return carry

    lax.fori_loop(0, RPS // ZROWS, _zcopy, 0)
    plsc.subcore_barrier()

    def _gather_start(j):
        b = lax.rem(j, NBUF)
        pltpu.async_copy(x_hbm.at[sd_v.at[0, j]], rows_v.at[b], sems.at[b])

    def _scatter_wait(j):
        if not _SCATTER_ON:
            return
        b = lax.rem(j, NBUF)
        pltpu.make_async_copy(rows_v.at[b], acc_sh.at[sd_v.at[1, j]],
                              ssems.at[b]).wait()

    def _pipe(j, carry):
        b = lax.rem(j, NBUF)
        pltpu.make_async_copy(x_hbm.at[sd_v.at[0, j]], rows_v.at[b],
                              sems.at[b]).wait()

        @pl.when(j >= DEPTH)
        def _():
            _scatter_wait(j - DEPTH)

        @pl.when(j + DEPTH < GC)
        def _():
            _gather_start(j + DEPTH)

        if _SCATTER_ON:
            pltpu.async_copy(rows_v.at[b], acc_sh.at[sd_v.at[1, j]],
                             ssems.at[b], add=True)
        return carry

    for g in range(NGRP):
        pltpu.sync_copy(sd_hbm.at[w, g], sd_v)
        for p in range(DEPTH):
            _gather_start(p)
        lax.fori_loop(0, GC, _pipe, 0)
        for p in range(DEPTH, 0, -1):
            _scatter_wait(GC - p)
    plsc.subcore_barrier()

    pltpu.sync_copy(acc_sh.at[pl.ds(base, RPS)], out_hbm.at[c, pl.ds(base, RPS)])


@functools.cache
def _sc_agg_kernel():
    return pl.kernel(
        _sc_agg_body,
        mesh=plsc.VectorSubcoreMesh(core_axis_name="c", subcore_axis_name="s"),
        out_type=jax.ShapeDtypeStruct((NC, NPAD, NDIM), jnp.float32),
        scratch_types=[
            pltpu.VMEM((2, GC, CHUNK), jnp.int32),
            pltpu.VMEM((NBUF, CHUNK, _GDIM), _GDT),
            pltpu.VMEM((ZROWS, NDIM), jnp.float32),
            pltpu.VMEM_SHARED((NPAD, NDIM), jnp.float32),
            pltpu.SemaphoreType.DMA((NBUF,)),
            pltpu.SemaphoreType.DMA((NBUF,)),
        ],
    )


NB = N // BLK


def _tc_body(x_ref, a0_ref, a1_ref, w1_ref, b1_ref, gamma_ref, beta_ref,
             w2_ref, b2_ref, wfc_ref, bfc_ref, z_ref,
             h1_s, stats_s, s_s, B_s, c_s):
    i = pl.program_id(0)

    @pl.when(i < NB)
    def _():
        h = x_ref[...] + a0_ref[...] + a1_ref[...]
        h1 = jnp.dot(h, w1_ref[...], preferred_element_type=jnp.float32)
        h1 = jnp.maximum(h1 + b1_ref[...], 0.0)
        h1_s[pl.ds(i * BLK, BLK), :] = h1
        blk = jnp.concatenate([jnp.sum(h1, axis=0, keepdims=True),
                               jnp.sum(h1 * h1, axis=0, keepdims=True)],
                              axis=0)

        @pl.when(i == 0)
        def _():
            stats_s[...] = blk

        @pl.when(i > 0)
        def _():
            stats_s[...] = stats_s[...] + blk

    @pl.when(i == NB)
    def _():
        inv_n = jnp.float32(1.0 / N)
        mean = stats_s[0:1, :] * inv_n
        var = stats_s[1:2, :] * inv_n - mean * mean
        s = gamma_ref[...] * lax.rsqrt(var + BN_EPS)
        t = beta_ref[...] - mean * s
        c = jnp.dot(
            jnp.dot(t, w2_ref[...], preferred_element_type=jnp.float32)
            + b2_ref[...],
            wfc_ref[...], preferred_element_type=jnp.float32) + bfc_ref[...]
        s_s[...] = s
        B_s[...] = jnp.dot(w2_ref[...], wfc_ref[...],
                           preferred_element_type=jnp.float32)
        c_s[...] = c

    @pl.when(i >= NB)
    def _():
        k = i - NB
        h1 = h1_s[pl.ds(k * BLK, BLK), :]
        z_ref[...] = jnp.dot(h1 * s_s[...], B_s[...],
                             preferred_element_type=jnp.float32) + c_s[...]


def _tc_fused(x, a0, a1, W1, b1r, gammar, betar, W2, b2r, Wfc, bfcr):
    blk_in = pl.BlockSpec((BLK, NDIM), lambda i: (jnp.minimum(i, NB - 1), 0))
    return pl.pallas_call(
        _tc_body,
        grid=(2 * NB,),
        in_specs=[
            blk_in, blk_in, blk_in,
            pl.BlockSpec((NDIM, HDL), lambda i: (0, 0)),
            pl.BlockSpec((1, HDL), lambda i: (0, 0)),
            pl.BlockSpec((1, HDL), lambda i: (0, 0)),
            pl.BlockSpec((1, HDL), lambda i: (0, 0)),
            pl.BlockSpec((HDL, HDL), lambda i: (0, 0)),
            pl.BlockSpec((1, HDL), lambda i: (0, 0)),
            pl.BlockSpec((HDL, R), lambda i: (0, 0)),
            pl.BlockSpec((1, R), lambda i: (0, 0)),
        ],
        out_specs=pl.BlockSpec((BLK, R),
                               lambda i: (jnp.maximum(i - NB, 0), 0)),
        out_shape=jax.ShapeDtypeStruct((N, R), jnp.float32),
        scratch_shapes=[
            pltpu.VMEM((N, HDL), jnp.float32),
            pltpu.VMEM((2, HDL), jnp.float32),
            pltpu.VMEM((1, HDL), jnp.float32),
            pltpu.VMEM((HDL, R), jnp.float32),
            pltpu.VMEM((1, R), jnp.float32),
        ],
    )(x, a0, a1, W1, b1r, gammar, betar, W2, b2r, Wfc, bfcr)


def kernel(x, edge_index, W1, b1, gamma, beta, W2, b2, Wfc, bfc):
    src = edge_index[0].astype(jnp.int32).reshape(NW, NGRP, GC, CHUNK)
    dst = edge_index[1].astype(jnp.int32).reshape(NW, NGRP, GC, CHUNK)
    sd = jnp.stack([src, dst], axis=2)
    x_sc = x[:, :_GDIM] if _GDIM < NDIM else x
    agg = _sc_agg_kernel()(x_sc.astype(_GDT), sd)
    return _tc_fused(x, agg[0, :N], agg[1, :N], W1, b1.reshape(1, HDL),
                     gamma.reshape(1, HDL), beta.reshape(1, HDL),
                     W2, b2.reshape(1, HDL), Wfc, bfc.reshape(1, R))


# speedup vs baseline: 1.1297x; 1.1297x over previous
import functools

import jax
import jax.numpy as jnp
from jax import lax
from jax.experimental import pallas as pl
from jax.experimental.pallas import tpu as pltpu
from jax.experimental.pallas import tpu_sc as plsc

N, E, NDIM, HDL, R = 10000, 320000, 128, 256, 128
BN_EPS = 1e-5

NC, NS = 2, 16
NW = NC * NS
CHUNK = 40
EPW = E // NW
NCHUNK = EPW // CHUNK
NGRP = 5
GC = NCHUNK // NGRP
NPAD = 10240
RPS = NPAD // NS
ZROWS = 16

BLK = 1000


NBUF = 6
DEPTH = NBUF - 2


def _sc_agg_body(x_hbm, src_hbm, dst_hbm, out_hbm, sd_v, rows_v, zero_v,
                 acc_sh, sems, ssems):
    c = lax.axis_index("c")
    s = lax.axis_index("s")
    w = c * NS + s

    zf = jnp.zeros((16,), jnp.float32)

    def _zfill(i, carry):
        r = i // (NDIM // 16)
        k = i % (NDIM // 16)
        zero_v[r, pl.ds(k * 16, 16)] = zf
        return carry

    lax.fori_loop(0, ZROWS * (NDIM // 16), _zfill, 0)

    base = s * RPS

    def _zcopy(t, carry):
        pltpu.sync_copy(zero_v, acc_sh.at[pl.ds(base + t * ZROWS, ZROWS)])
        return carry

    lax.fori_loop(0, RPS // ZROWS, _zcopy, 0)
    plsc.subcore_barrier()

    def _gather_start(j):
        b = lax.rem(j, NBUF)
        pltpu.async_copy(x_hbm.at[sd_v.at[0, j]], rows_v.at[b], sems.at[b])

    def _scatter_wait(j):
        b = lax.rem(j, NBUF)
        pltpu.make_async_copy(rows_v.at[b], acc_sh.at[sd_v.at[1, j]],
                              ssems.at[b]).wait()

    def _pipe(j, carry):
        b = lax.rem(j, NBUF)
        pltpu.make_async_copy(x_hbm.at[sd_v.at[0, j]], rows_v.at[b],
                              sems.at[b]).wait()

        @pl.when(j >= DEPTH)
        def _():
            _scatter_wait(j - DEPTH)

        @pl.when(j + DEPTH < GC)
        def _():
            _gather_start(j + DEPTH)

        pltpu.async_copy(rows_v.at[b], acc_sh.at[sd_v.at[1, j]],
                         ssems.at[b], add=True)
        return carry

    for g in range(NGRP):
        pltpu.sync_copy(src_hbm.at[w, g], sd_v.at[0])
        pltpu.sync_copy(dst_hbm.at[w, g], sd_v.at[1])
        for p in range(DEPTH):
            _gather_start(p)
        lax.fori_loop(0, GC, _pipe, 0)
        for p in range(DEPTH, 0, -1):
            _scatter_wait(GC - p)
    plsc.subcore_barrier()

    pltpu.sync_copy(acc_sh.at[pl.ds(base, RPS)], out_hbm.at[c, pl.ds(base, RPS)])


@functools.cache
def _sc_agg_kernel():
    return pl.kernel(
        _sc_agg_body,
        mesh=plsc.VectorSubcoreMesh(core_axis_name="c", subcore_axis_name="s"),
        out_type=jax.ShapeDtypeStruct((NC, NPAD, NDIM), jnp.float32),
        scratch_types=[
            pltpu.VMEM((2, GC, CHUNK), jnp.int32),
            pltpu.VMEM((NBUF, CHUNK, NDIM), jnp.float32),
            pltpu.VMEM((ZROWS, NDIM), jnp.float32),
            pltpu.VMEM_SHARED((NPAD, NDIM), jnp.float32),
            pltpu.SemaphoreType.DMA((NBUF,)),
            pltpu.SemaphoreType.DMA((NBUF,)),
        ],
    )


NB = N // BLK


def _tc_body(x_ref, a0_ref, a1_ref, w1_ref, b1_ref, gamma_ref, beta_ref,
             w2_ref, b2_ref, wfc_ref, bfc_ref, z_ref,
             h1_s, stats_s, s_s, B_s, c_s):
    i = pl.program_id(0)

    @pl.when(i < NB)
    def _():
        h = x_ref[...] + a0_ref[0] + a1_ref[0]
        h1 = jnp.dot(h, w1_ref[...], preferred_element_type=jnp.float32)
        h1 = jnp.maximum(h1 + b1_ref[...], 0.0)
        h1_s[pl.ds(i * BLK, BLK), :] = h1
        blk = jnp.concatenate([jnp.sum(h1, axis=0, keepdims=True),
                               jnp.sum(h1 * h1, axis=0, keepdims=True)],
                              axis=0)

        @pl.when(i == 0)
        def _():
            stats_s[...] = blk

        @pl.when(i > 0)
        def _():
            stats_s[...] = stats_s[...] + blk

    @pl.when(i == NB)
    def _():
        inv_n = jnp.float32(1.0 / N)
        mean = stats_s[0:1, :] * inv_n
        var = stats_s[1:2, :] * inv_n - mean * mean
        s = gamma_ref[...] * lax.rsqrt(var + BN_EPS)
        t = beta_ref[...] - mean * s
        c = jnp.dot(
            jnp.dot(t, w2_ref[...], preferred_element_type=jnp.float32)
            + b2_ref[...],
            wfc_ref[...], preferred_element_type=jnp.float32) + bfc_ref[...]
        s_s[...] = s
        B_s[...] = jnp.dot(w2_ref[...], wfc_ref[...],
                           preferred_element_type=jnp.float32)
        c_s[...] = c

    @pl.when(i >= NB)
    def _():
        k = i - NB
        h1 = h1_s[pl.ds(k * BLK, BLK), :]
        z_ref[...] = jnp.dot(h1 * s_s[...], B_s[...],
                             preferred_element_type=jnp.float32) + c_s[...]


def _tc_fused(x, agg, W1, b1r, gammar, betar, W2, b2r, Wfc, bfcr):
    blk_in = pl.BlockSpec((BLK, NDIM), lambda i: (jnp.minimum(i, NB - 1), 0))
    a_spec = lambda cidx: pl.BlockSpec(
        (1, BLK, NDIM), lambda i: (cidx, jnp.minimum(i, NB - 1), 0))
    return pl.pallas_call(
        _tc_body,
        grid=(2 * NB,),
        in_specs=[
            blk_in, a_spec(0), a_spec(1),
            pl.BlockSpec((NDIM, HDL), lambda i: (0, 0)),
            pl.BlockSpec((1, HDL), lambda i: (0, 0)),
            pl.BlockSpec((1, HDL), lambda i: (0, 0)),
            pl.BlockSpec((1, HDL), lambda i: (0, 0)),
            pl.BlockSpec((HDL, HDL), lambda i: (0, 0)),
            pl.BlockSpec((1, HDL), lambda i: (0, 0)),
            pl.BlockSpec((HDL, R), lambda i: (0, 0)),
            pl.BlockSpec((1, R), lambda i: (0, 0)),
        ],
        out_specs=pl.BlockSpec((BLK, R),
                               lambda i: (jnp.maximum(i - NB, 0), 0)),
        out_shape=jax.ShapeDtypeStruct((N, R), jnp.float32),
        scratch_shapes=[
            pltpu.VMEM((N, HDL), jnp.float32),
            pltpu.VMEM((2, HDL), jnp.float32),
            pltpu.VMEM((1, HDL), jnp.float32),
            pltpu.VMEM((HDL, R), jnp.float32),
            pltpu.VMEM((1, R), jnp.float32),
        ],
    )(x, agg, agg, W1, b1r, gammar, betar, W2, b2r, Wfc, bfcr)


def kernel(x, edge_index, W1, b1, gamma, beta, W2, b2, Wfc, bfc):
    src = edge_index[0].astype(jnp.int32).reshape(NW, NGRP, GC, CHUNK)
    dst = edge_index[1].astype(jnp.int32).reshape(NW, NGRP, GC, CHUNK)
    agg = _sc_agg_kernel()(x, src, dst)
    return _tc_fused(x, agg, W1, b1.reshape(1, HDL),
                     gamma.reshape(1, HDL), beta.reshape(1, HDL),
                     W2, b2.reshape(1, HDL), Wfc, bfc.reshape(1, R))


# speedup vs baseline: 1.1611x; 1.0278x over previous
import functools

import jax
import jax.numpy as jnp
from jax import lax
from jax.experimental import pallas as pl
from jax.experimental.pallas import tpu as pltpu
from jax.experimental.pallas import tpu_sc as plsc

N, E, NDIM, HDL, R = 10000, 320000, 128, 256, 128
BN_EPS = 1e-5

NC, NS = 2, 16
NW = NC * NS
CHUNK = 40
EPW = E // NW
NCHUNK = EPW // CHUNK
NGRP = 5
GC = NCHUNK // NGRP
NPAD = 10240
RPS = NPAD // NS
ZROWS = 16

BLK = 2000


NBUF = 6
DEPTH = NBUF - 2


def _sc_agg_body(x_hbm, src_hbm, dst_hbm, out_hbm, sd_v, rows_v, zero_v,
                 acc_sh, sems, ssems):
    c = lax.axis_index("c")
    s = lax.axis_index("s")
    w = c * NS + s

    zf = jnp.zeros((16,), jnp.float32)

    def _zfill(i, carry):
        r = i // (NDIM // 16)
        k = i % (NDIM // 16)
        zero_v[r, pl.ds(k * 16, 16)] = zf
        return carry

    lax.fori_loop(0, ZROWS * (NDIM // 16), _zfill, 0)

    base = s * RPS

    def _zcopy(t, carry):
        pltpu.sync_copy(zero_v, acc_sh.at[pl.ds(base + t * ZROWS, ZROWS)])
        return carry

    lax.fori_loop(0, RPS // ZROWS, _zcopy, 0)
    plsc.subcore_barrier()

    def _gather_start(j):
        b = lax.rem(j, NBUF)
        pltpu.async_copy(x_hbm.at[sd_v.at[0, j]], rows_v.at[b], sems.at[b])

    def _scatter_wait(j):
        b = lax.rem(j, NBUF)
        pltpu.make_async_copy(rows_v.at[b], acc_sh.at[sd_v.at[1, j]],
                              ssems.at[b]).wait()

    def _pipe(j, carry):
        b = lax.rem(j, NBUF)
        pltpu.make_async_copy(x_hbm.at[sd_v.at[0, j]], rows_v.at[b],
                              sems.at[b]).wait()

        @pl.when(j >= DEPTH)
        def _():
            _scatter_wait(j - DEPTH)

        @pl.when(j + DEPTH < GC)
        def _():
            _gather_start(j + DEPTH)

        pltpu.async_copy(rows_v.at[b], acc_sh.at[sd_v.at[1, j]],
                         ssems.at[b], add=True)
        return carry

    for g in range(NGRP):
        pltpu.sync_copy(src_hbm.at[w, g], sd_v.at[0])
        pltpu.sync_copy(dst_hbm.at[w, g], sd_v.at[1])
        for p in range(DEPTH):
            _gather_start(p)
        lax.fori_loop(0, GC, _pipe, 0)
        for p in range(DEPTH, 0, -1):
            _scatter_wait(GC - p)
    plsc.subcore_barrier()

    pltpu.sync_copy(acc_sh.at[pl.ds(base, RPS)], out_hbm.at[c, pl.ds(base, RPS)])


@functools.cache
def _sc_agg_kernel():
    return pl.kernel(
        _sc_agg_body,
        mesh=plsc.VectorSubcoreMesh(core_axis_name="c", subcore_axis_name="s"),
        out_type=jax.ShapeDtypeStruct((NC, NPAD, NDIM), jnp.float32),
        scratch_types=[
            pltpu.VMEM((2, GC, CHUNK), jnp.int32),
            pltpu.VMEM((NBUF, CHUNK, NDIM), jnp.float32),
            pltpu.VMEM((ZROWS, NDIM), jnp.float32),
            pltpu.VMEM_SHARED((NPAD, NDIM), jnp.float32),
            pltpu.SemaphoreType.DMA((NBUF,)),
            pltpu.SemaphoreType.DMA((NBUF,)),
        ],
    )


NB = N // BLK


def _tc_body(x_ref, a0_ref, a1_ref, w1_ref, b1_ref, gamma_ref, beta_ref,
             w2_ref, b2_ref, wfc_ref, bfc_ref, z_ref,
             h1_s, stats_s, s_s, B_s, c_s):
    i = pl.program_id(0)

    @pl.when(i < NB)
    def _():
        h = x_ref[...] + a0_ref[0] + a1_ref[0]
        h1 = jnp.dot(h, w1_ref[...], preferred_element_type=jnp.float32)
        h1 = jnp.maximum(h1 + b1_ref[...], 0.0)
        h1_s[pl.ds(i * BLK, BLK), :] = h1
        blk = jnp.concatenate([jnp.sum(h1, axis=0, keepdims=True),
                               jnp.sum(h1 * h1, axis=0, keepdims=True)],
                              axis=0)

        @pl.when(i == 0)
        def _():
            stats_s[...] = blk

        @pl.when(i > 0)
        def _():
            stats_s[...] = stats_s[...] + blk

    @pl.when(i == NB)
    def _():
        inv_n = jnp.float32(1.0 / N)
        mean = stats_s[0:1, :] * inv_n
        var = stats_s[1:2, :] * inv_n - mean * mean
        s = gamma_ref[...] * lax.rsqrt(var + BN_EPS)
        t = beta_ref[...] - mean * s
        c = jnp.dot(
            jnp.dot(t, w2_ref[...], preferred_element_type=jnp.float32)
            + b2_ref[...],
            wfc_ref[...], preferred_element_type=jnp.float32) + bfc_ref[...]
        s_s[...] = s
        B_s[...] = jnp.dot(w2_ref[...], wfc_ref[...],
                           preferred_element_type=jnp.float32)
        c_s[...] = c

    @pl.when(i >= NB)
    def _():
        k = i - NB
        h1 = h1_s[pl.ds(k * BLK, BLK), :]
        z_ref[...] = jnp.dot(h1 * s_s[...], B_s[...],
                             preferred_element_type=jnp.float32) + c_s[...]


def _tc_fused(x, agg, W1, b1r, gammar, betar, W2, b2r, Wfc, bfcr):
    blk_in = pl.BlockSpec((BLK, NDIM), lambda i: (jnp.minimum(i, NB - 1), 0))
    a_spec = lambda cidx: pl.BlockSpec(
        (1, BLK, NDIM), lambda i: (cidx, jnp.minimum(i, NB - 1), 0))
    return pl.pallas_call(
        _tc_body,
        grid=(2 * NB,),
        in_specs=[
            blk_in, a_spec(0), a_spec(1),
            pl.BlockSpec((NDIM, HDL), lambda i: (0, 0)),
            pl.BlockSpec((1, HDL), lambda i: (0, 0)),
            pl.BlockSpec((1, HDL), lambda i: (0, 0)),
            pl.BlockSpec((1, HDL), lambda i: (0, 0)),
            pl.BlockSpec((HDL, HDL), lambda i: (0, 0)),
            pl.BlockSpec((1, HDL), lambda i: (0, 0)),
            pl.BlockSpec((HDL, R), lambda i: (0, 0)),
            pl.BlockSpec((1, R), lambda i: (0, 0)),
        ],
        out_specs=pl.BlockSpec((BLK, R),
                               lambda i: (jnp.maximum(i - NB, 0), 0)),
        out_shape=jax.ShapeDtypeStruct((N, R), jnp.float32),
        scratch_shapes=[
            pltpu.VMEM((N, HDL), jnp.float32),
            pltpu.VMEM((2, HDL), jnp.float32),
            pltpu.VMEM((1, HDL), jnp.float32),
            pltpu.VMEM((HDL, R), jnp.float32),
            pltpu.VMEM((1, R), jnp.float32),
        ],
    )(x, agg, agg, W1, b1r, gammar, betar, W2, b2r, Wfc, bfcr)


def kernel(x, edge_index, W1, b1, gamma, beta, W2, b2, Wfc, bfc):
    src = edge_index[0].astype(jnp.int32).reshape(NW, NGRP, GC, CHUNK)
    dst = edge_index[1].astype(jnp.int32).reshape(NW, NGRP, GC, CHUNK)
    agg = _sc_agg_kernel()(x, src, dst)
    return _tc_fused(x, agg, W1, b1.reshape(1, HDL),
                     gamma.reshape(1, HDL), beta.reshape(1, HDL),
                     W2, b2.reshape(1, HDL), Wfc, bfc.reshape(1, R))


# speedup vs baseline: 1.1765x; 1.0132x over previous
import functools

import jax
import jax.numpy as jnp
from jax import lax
from jax.experimental import pallas as pl
from jax.experimental.pallas import tpu as pltpu
from jax.experimental.pallas import tpu_sc as plsc

N, E, NDIM, HDL, R = 10000, 320000, 128, 256, 128
BN_EPS = 1e-5

NC, NS = 2, 16
NW = NC * NS
CHUNK = 40
EPW = E // NW
NCHUNK = EPW // CHUNK
NGRP = 5
GC = NCHUNK // NGRP
NPAD = 10240
RPS = NPAD // NS
ZROWS = 16

BLK = 5000


NBUF = 6
DEPTH = NBUF - 2


def _sc_agg_body(x_hbm, src_hbm, dst_hbm, out_hbm, sd_v, rows_v, zero_v,
                 acc_sh, sems, ssems):
    c = lax.axis_index("c")
    s = lax.axis_index("s")
    w = c * NS + s

    zf = jnp.zeros((16,), jnp.float32)

    def _zfill(i, carry):
        r = i // (NDIM // 16)
        k = i % (NDIM // 16)
        zero_v[r, pl.ds(k * 16, 16)] = zf
        return carry

    lax.fori_loop(0, ZROWS * (NDIM // 16), _zfill, 0)

    base = s * RPS

    def _zcopy(t, carry):
        pltpu.sync_copy(zero_v, acc_sh.at[pl.ds(base + t * ZROWS, ZROWS)])
        return carry

    lax.fori_loop(0, RPS // ZROWS, _zcopy, 0)
    plsc.subcore_barrier()

    def _gather_start(j):
        b = lax.rem(j, NBUF)
        pltpu.async_copy(x_hbm.at[sd_v.at[0, j]], rows_v.at[b], sems.at[b])

    def _scatter_wait(j):
        b = lax.rem(j, NBUF)
        pltpu.make_async_copy(rows_v.at[b], acc_sh.at[sd_v.at[1, j]],
                              ssems.at[b]).wait()

    def _pipe(j, carry):
        b = lax.rem(j, NBUF)
        pltpu.make_async_copy(x_hbm.at[sd_v.at[0, j]], rows_v.at[b],
                              sems.at[b]).wait()

        @pl.when(j >= DEPTH)
        def _():
            _scatter_wait(j - DEPTH)

        @pl.when(j + DEPTH < GC)
        def _():
            _gather_start(j + DEPTH)

        pltpu.async_copy(rows_v.at[b], acc_sh.at[sd_v.at[1, j]],
                         ssems.at[b], add=True)
        return carry

    for g in range(NGRP):
        pltpu.sync_copy(src_hbm.at[w, g], sd_v.at[0])
        pltpu.sync_copy(dst_hbm.at[w, g], sd_v.at[1])
        for p in range(DEPTH):
            _gather_start(p)
        lax.fori_loop(0, GC, _pipe, 0)
        for p in range(DEPTH, 0, -1):
            _scatter_wait(GC - p)
    plsc.subcore_barrier()

    pltpu.sync_copy(acc_sh.at[pl.ds(base, RPS)], out_hbm.at[c, pl.ds(base, RPS)])


@functools.cache
def _sc_agg_kernel():
    return pl.kernel(
        _sc_agg_body,
        mesh=plsc.VectorSubcoreMesh(core_axis_name="c", subcore_axis_name="s"),
        out_type=jax.ShapeDtypeStruct((NC, NPAD, NDIM), jnp.float32),
        scratch_types=[
            pltpu.VMEM((2, GC, CHUNK), jnp.int32),
            pltpu.VMEM((NBUF, CHUNK, NDIM), jnp.float32),
            pltpu.VMEM((ZROWS, NDIM), jnp.float32),
            pltpu.VMEM_SHARED((NPAD, NDIM), jnp.float32),
            pltpu.SemaphoreType.DMA((NBUF,)),
            pltpu.SemaphoreType.DMA((NBUF,)),
        ],
    )


NB = N // BLK


def _tc_body(x_ref, a0_ref, a1_ref, w1_ref, b1_ref, gamma_ref, beta_ref,
             w2_ref, b2_ref, wfc_ref, bfc_ref, z_ref,
             h1_s, stats_s, s_s, B_s, c_s):
    i = pl.program_id(0)

    @pl.when(i < NB)
    def _():
        h = x_ref[...] + a0_ref[0] + a1_ref[0]
        h1 = jnp.dot(h, w1_ref[...], preferred_element_type=jnp.float32)
        h1 = jnp.maximum(h1 + b1_ref[...], 0.0)
        h1_s[pl.ds(i * BLK, BLK), :] = h1
        blk = jnp.concatenate([jnp.sum(h1, axis=0, keepdims=True),
                               jnp.sum(h1 * h1, axis=0, keepdims=True)],
                              axis=0)

        @pl.when(i == 0)
        def _():
            stats_s[...] = blk

        @pl.when(i > 0)
        def _():
            stats_s[...] = stats_s[...] + blk

    @pl.when(i == NB)
    def _():
        inv_n = jnp.float32(1.0 / N)
        mean = stats_s[0:1, :] * inv_n
        var = stats_s[1:2, :] * inv_n - mean * mean
        s = gamma_ref[...] * lax.rsqrt(var + BN_EPS)
        t = beta_ref[...] - mean * s
        c = jnp.dot(
            jnp.dot(t, w2_ref[...], preferred_element_type=jnp.float32)
            + b2_ref[...],
            wfc_ref[...], preferred_element_type=jnp.float32) + bfc_ref[...]
        s_s[...] = s
        B_s[...] = jnp.dot(w2_ref[...], wfc_ref[...],
                           preferred_element_type=jnp.float32)
        c_s[...] = c

    @pl.when(i >= NB)
    def _():
        k = i - NB
        h1 = h1_s[pl.ds(k * BLK, BLK), :]
        z_ref[...] = jnp.dot(h1 * s_s[...], B_s[...],
                             preferred_element_type=jnp.float32) + c_s[...]


def _tc_fused(x, agg, W1, b1r, gammar, betar, W2, b2r, Wfc, bfcr):
    blk_in = pl.BlockSpec((BLK, NDIM), lambda i: (jnp.minimum(i, NB - 1), 0))
    a_spec = lambda cidx: pl.BlockSpec(
        (1, BLK, NDIM), lambda i: (cidx, jnp.minimum(i, NB - 1), 0))
    return pl.pallas_call(
        _tc_body,
        grid=(2 * NB,),
        in_specs=[
            blk_in, a_spec(0), a_spec(1),
            pl.BlockSpec((NDIM, HDL), lambda i: (0, 0)),
            pl.BlockSpec((1, HDL), lambda i: (0, 0)),
            pl.BlockSpec((1, HDL), lambda i: (0, 0)),
            pl.BlockSpec((1, HDL), lambda i: (0, 0)),
            pl.BlockSpec((HDL, HDL), lambda i: (0, 0)),
            pl.BlockSpec((1, HDL), lambda i: (0, 0)),
            pl.BlockSpec((HDL, R), lambda i: (0, 0)),
            pl.BlockSpec((1, R), lambda i: (0, 0)),
        ],
        out_specs=pl.BlockSpec((BLK, R),
                               lambda i: (jnp.maximum(i - NB, 0), 0)),
        out_shape=jax.ShapeDtypeStruct((N, R), jnp.float32),
        scratch_shapes=[
            pltpu.VMEM((N, HDL), jnp.float32),
            pltpu.VMEM((2, HDL), jnp.float32),
            pltpu.VMEM((1, HDL), jnp.float32),
            pltpu.VMEM((HDL, R), jnp.float32),
            pltpu.VMEM((1, R), jnp.float32),
        ],
    )(x, agg, agg, W1, b1r, gammar, betar, W2, b2r, Wfc, bfcr)


def kernel(x, edge_index, W1, b1, gamma, beta, W2, b2, Wfc, bfc):
    src = edge_index[0].astype(jnp.int32).reshape(NW, NGRP, GC, CHUNK)
    dst = edge_index[1].astype(jnp.int32).reshape(NW, NGRP, GC, CHUNK)
    agg = _sc_agg_kernel()(x, src, dst)
    return _tc_fused(x, agg, W1, b1.reshape(1, HDL),
                     gamma.reshape(1, HDL), beta.reshape(1, HDL),
                     W2, b2.reshape(1, HDL), Wfc, bfc.reshape(1, R))
